# baseline (speedup 1.0000x reference)
import functools

import jax
import jax.numpy as jnp
from jax import lax
from jax.experimental import pallas as pl
from jax.experimental.pallas import tpu as pltpu
from jax.experimental.pallas import tpu_sc as plsc

N = 10000
EDG = 320000
D = 128
NCONV = 13
NG = 64
NC = 2
NS = 16
NW = NC * NS
L = 16
CK = 64
CH = 160
DCK = 128
DCH = 80
EP = NW * CH * CK
RSUB = 624
RTAIL = N - NS * RSUB
NPAD = NW * 320
HID = NCONV * 16


def _edge_mlp_body(ea_ref, w1_ref, b1_ref, w2_ref, b2_ref, o_ref):
    a = ea_ref[...]
    h = jnp.maximum(
        jnp.dot(a, w1_ref[...], preferred_element_type=jnp.float32) + b1_ref[...],
        0.0)
    z = jnp.dot(h, w2_ref[...], preferred_element_type=jnp.float32) + b2_ref[...]
    o_ref[...] = jax.nn.sigmoid(z)


def _edge_mlp(edge_attr, w1, b1, w2, b2):
    blk = 2000
    return pl.pallas_call(
        _edge_mlp_body,
        grid=(EDG // blk,),
        in_specs=[
            pl.BlockSpec((blk, 16), lambda i: (i, 0)),
            pl.BlockSpec((16, HID), lambda i: (0, 0)),
            pl.BlockSpec((1, HID), lambda i: (0, 0)),
            pl.BlockSpec((HID, 16), lambda i: (0, 0)),
            pl.BlockSpec((1, 16), lambda i: (0, 0)),
        ],
        out_specs=pl.BlockSpec((blk, 16), lambda i: (i, 0)),
        out_shape=jax.ShapeDtypeStruct((EDG, 16), jnp.float32),
    )(edge_attr, w1, b1, w2, b2)


def _dinv_body(degp_ref, dinv_ref, dinv2_ref):
    d = 1.0 + degp_ref[0][:, 0:16] + degp_ref[1][:, 0:16]
    dinv_ref[...] = lax.rsqrt(d)
    dinv2_ref[...] = 1.0 / d


def _dinv(degp):
    return pl.pallas_call(
        _dinv_body,
        out_shape=[
            jax.ShapeDtypeStruct((N, 16), jnp.float32),
            jax.ShapeDtypeStruct((N, 16), jnp.float32),
        ],
    )(degp)


def _pre0_body(x_ref, w_ref, o_ref):
    o_ref[...] = jnp.dot(x_ref[...], w_ref[...],
                         preferred_element_type=jnp.float32)


def _pre0(x, w):
    return pl.pallas_call(
        _pre0_body,
        out_shape=jax.ShapeDtypeStruct((N, D), jnp.float32),
    )(x, w)


def _asm(p_ref, xl_ref, d2_ref, b_ref, ci):
    d2c = d2_ref[...][:, ci:ci + 1]
    return p_ref[0] + p_ref[1] + d2c * xl_ref[...] + b_ref[...]


def _bn_relu_mm(xx, w_ref):
    m = jnp.mean(xx, axis=0, keepdims=True)
    xc = xx - m
    v = jnp.mean(xc * xc, axis=0, keepdims=True)
    h = jnp.maximum(xc * lax.rsqrt(v + 1e-5), 0.0)
    return jnp.dot(h, w_ref[...], preferred_element_type=jnp.float32)


def _pre_plain(part, xl, d2, b, wnext, ci):
    def body(p_ref, xl_ref, d2_ref, b_ref, w_ref, o_ref):
        o_ref[...] = _bn_relu_mm(_asm(p_ref, xl_ref, d2_ref, b_ref, ci), w_ref)
    return pl.pallas_call(
        body, out_shape=jax.ShapeDtypeStruct((N, D), jnp.float32),
    )(part, xl, d2, b, wnext)


def _pre_first(part, xl, d2, b, wnext, ci):
    def body(p_ref, xl_ref, d2_ref, b_ref, w_ref, o_ref, s_ref):
        x1 = _asm(p_ref, xl_ref, d2_ref, b_ref, ci)
        s_ref[...] = x1
        o_ref[...] = _bn_relu_mm(x1, w_ref)
    return pl.pallas_call(
        body,
        out_shape=[jax.ShapeDtypeStruct((N, D), jnp.float32),
                   jax.ShapeDtypeStruct((N, D), jnp.float32)],
    )(part, xl, d2, b, wnext)


def _pre_bound(part, xl, d2, b, wnext, s_in, ci):
    def body(p_ref, xl_ref, d2_ref, b_ref, w_ref, sin_ref, o_ref, sout_ref):
        xa = _asm(p_ref, xl_ref, d2_ref, b_ref, ci)
        s = xa + sin_ref[...]
        sout_ref[...] = sin_ref[...] + s
        o_ref[...] = _bn_relu_mm(s, w_ref)
    return pl.pallas_call(
        body,
        out_shape=[jax.ShapeDtypeStruct((N, D), jnp.float32),
                   jax.ShapeDtypeStruct((N, D), jnp.float32)],
    )(part, xl, d2, b, wnext, s_in)


def _final_asm(part, xl, d2, b, s_in, ci):
    def body(p_ref, xl_ref, d2_ref, b_ref, sin_ref, o_ref):
        xa = _asm(p_ref, xl_ref, d2_ref, b_ref, ci)
        s = xa + sin_ref[...]
        o_ref[pl.ds(0, N), :] = jnp.maximum(s, 0.0)
        o_ref[pl.ds(N, NPAD - N), :] = jnp.full((NPAD - N, D), -1.0,
                                                jnp.float32)
    return pl.pallas_call(
        body, out_shape=jax.ShapeDtypeStruct((NPAD, D), jnp.float32),
    )(part, xl, d2, b, s_in)


def _logits(poolp, glob, lw, lb):
    def body(pp_ref, g_ref, lw_ref, lb_ref, o_ref):
        pooled = jnp.max(pp_ref[...], axis=0)
        pooled = jnp.where(pooled < 0.0, 0.0, pooled)
        o_ref[...] = (
            jnp.dot(pooled, lw_ref[0:D, :], preferred_element_type=jnp.float32)
            + jnp.dot(g_ref[...], lw_ref[D:D + 16, :],
                      preferred_element_type=jnp.float32)
            + lb_ref[...])
    return pl.pallas_call(
        body, out_shape=jax.ShapeDtypeStruct((NG, 10), jnp.float32),
    )(poolp, glob, lw, lb)


@functools.cache
def _sc_mesh():
    return plsc.VectorSubcoreMesh(core_axis_name="c", subcore_axis_name="s")


@functools.cache
def _deg_kernel():
    @functools.partial(
        pl.kernel,
        out_type=jax.ShapeDtypeStruct((NC, N, D), jnp.float32),
        mesh=_sc_mesh(),
        compiler_params=pltpu.CompilerParams(needs_layout_passes=False),
        scratch_types=[
            pltpu.VMEM((DCH, DCK), jnp.int32),
            pltpu.VMEM((DCK * 16,), jnp.float32),
            pltpu.VMEM((DCK, D), jnp.float32),
            pltpu.VMEM((DCK, D), jnp.float32),
            pltpu.VMEM_SHARED((N, D), jnp.float32),
            pltpu.SemaphoreType.DMA((2,)),
        ],
    )
    def k(w16_hbm, dst_hbm, out_hbm, dst_v, wbuf, e0, e1, acc, ssem):
        ebufs = (e0, e1)
        cid = lax.axis_index("c")
        sid = lax.axis_index("s")
        wid = sid * NC + cid
        z16 = jnp.zeros((L,), jnp.float32)

        def zrow(i, _):
            for v in range(D // L):
                e0[i, pl.ds(v * L, L)] = z16
                e1[i, pl.ds(v * L, L)] = z16
            return 0
        lax.fori_loop(0, DCK, zrow, 0)
        for off, sz in ((0, 128), (128, 128), (256, 128), (384, 128),
                        (512, 112)):
            pltpu.sync_copy(e0.at[pl.ds(0, sz)],
                            acc.at[pl.ds(sid * RSUB + off, sz)])

        @pl.when(sid == 0)
        def _ztail():
            pltpu.sync_copy(e0.at[pl.ds(0, RTAIL)],
                            acc.at[pl.ds(NS * RSUB, RTAIL)])
        plsc.subcore_barrier()
        pltpu.sync_copy(dst_hbm.at[wid], dst_v)

        def gstep(g, _):
            for b in range(2):
                j = 2 * g + b
                pltpu.sync_copy(w16_hbm.at[wid, j], wbuf)

                @pl.when(j >= 2)
                def _ws():
                    pltpu.make_async_copy(ebufs[b], acc.at[dst_v.at[0]],
                                          ssem.at[b]).wait()

                def row(r, _):
                    ebufs[b][r, pl.ds(0, L)] = wbuf[pl.ds(r * 16, L)]
                    return 0
                lax.fori_loop(0, DCK, row, 0)
                pltpu.async_copy(ebufs[b], acc.at[dst_v.at[j]],
                                 ssem.at[b], add=True)
            return 0
        lax.fori_loop(0, DCH // 2, gstep, 0)
        for b in range(2):
            pltpu.make_async_copy(ebufs[b], acc.at[dst_v.at[0]],
                                  ssem.at[b]).wait()
        plsc.subcore_barrier()
        pltpu.sync_copy(acc.at[pl.ds(sid * RSUB, RSUB)],
                        out_hbm.at[cid, pl.ds(sid * RSUB, RSUB)])

        @pl.when(sid == 0)
        def _dtail():
            pltpu.sync_copy(acc.at[pl.ds(NS * RSUB, RTAIL)],
                            out_hbm.at[cid, pl.ds(NS * RSUB, RTAIL)])
    return k


@functools.cache
def _msg_kernel():
    @functools.partial(
        pl.kernel,
        out_type=jax.ShapeDtypeStruct((NC, N, D), jnp.float32),
        mesh=_sc_mesh(),
        compiler_params=pltpu.CompilerParams(needs_layout_passes=False),
        scratch_types=[
            pltpu.VMEM((2, 8, CK), jnp.int32),
            pltpu.VMEM((2, 8, CK), jnp.int32),
            pltpu.VMEM((2, 8, CK), jnp.float32),
            pltpu.VMEM((N,), jnp.float32),
            pltpu.VMEM((CK,), jnp.float32),
            pltpu.VMEM((CK, D), jnp.float32),
            pltpu.VMEM((CK, D), jnp.float32),
            pltpu.VMEM((CK, D), jnp.float32),
            pltpu.VMEM((CK, D), jnp.float32),
            pltpu.VMEM_SHARED((N, D), jnp.float32),
            pltpu.SemaphoreType.DMA((4,)),
            pltpu.SemaphoreType.DMA((4,)),
        ],
    )
    def k(xl_hbm, src_hbm, dst_hbm, w_hbm, dinv_hbm, out_hbm,
          src_v, dst_v, w_v, dinv_v, nrm_v, g0, g1, g2, g3, acc, gsem, ssem):
        gbufs = (g0, g1, g2, g3)
        cid = lax.axis_index("c")
        sid = lax.axis_index("s")
        wid = sid * NC + cid
        z16 = jnp.zeros((L,), jnp.float32)

        def zrow(i, _):
            for v in range(D // L):
                g0[i, pl.ds(v * L, L)] = z16
            return 0
        lax.fori_loop(0, CK, zrow, 0)
        for off in (0, 64, 128, 192, 256, 320, 384, 448, 512):
            pltpu.sync_copy(g0.at[pl.ds(0, 64)],
                            acc.at[pl.ds(sid * RSUB + off, 64)])
        pltpu.sync_copy(g0.at[pl.ds(0, 48)],
                        acc.at[pl.ds(sid * RSUB + 576, 48)])

        @pl.when(sid == 0)
        def _ztail():
            pltpu.sync_copy(g0.at[pl.ds(0, RTAIL)],
                            acc.at[pl.ds(NS * RSUB, RTAIL)])
        plsc.subcore_barrier()
        pltpu.sync_copy(dinv_hbm, dinv_v)

        def stage(grp_i, slot):
            base = pl.multiple_of(grp_i * 8, 8)
            pltpu.sync_copy(src_hbm.at[wid, pl.ds(base, 8)], src_v.at[slot])
            pltpu.sync_copy(dst_hbm.at[wid, pl.ds(base, 8)], dst_v.at[slot])
            pltpu.sync_copy(w_hbm.at[wid, pl.ds(base, 8)], w_v.at[slot])

        stage(0, 0)
        for b3 in range(3):
            pltpu.async_copy(xl_hbm.at[src_v.at[0, b3]], gbufs[b3],
                             gsem.at[b3])

        def gstep(g, _):
            for b in range(4):
                j = 4 * g + b
                pltpu.make_async_copy(xl_hbm.at[src_v.at[0, 0]], gbufs[b],
                                      gsem.at[b]).wait()
                slot = (j // 8) % 2
                r = j % 8
                for v in range(CK // L):
                    sl = pl.ds(v * L, L)
                    si = src_v[slot, r, sl]
                    di = dst_v[slot, r, sl]
                    nrm_v[sl] = (plsc.load_gather(dinv_v, [si])
                                 * w_v[slot, r, sl]
                                 * plsc.load_gather(dinv_v, [di]))

                def row(rr, _):
                    nb = plsc.load_gather(nrm_v,
                                          [jnp.full((L,), rr, jnp.int32)])
                    for v in range(D // L):
                        sl = pl.ds(v * L, L)
                        gbufs[b][rr, sl] = gbufs[b][rr, sl] * nb
                    return 0
                lax.fori_loop(0, CK, row, 0)
                pltpu.async_copy(gbufs[b], acc.at[dst_v.at[slot, r]],
                                 ssem.at[b], add=True)
                bn = (b + 3) % 4

                @pl.when(j >= 1)
                def _ws():
                    pltpu.make_async_copy(gbufs[bn], acc.at[dst_v.at[0, 0]],
                                          ssem.at[bn]).wait()
                j3 = j + 3
                if b == 1:
                    @pl.when((g % 2 == 1) & (j3 < CH))
                    def _st():
                        stage(j3 // 8, (j3 // 8) % 2)

                @pl.when(j3 < CH)
                def _ig():
                    pltpu.async_copy(
                        xl_hbm.at[src_v.at[(j3 // 8) % 2, j3 % 8]],
                        gbufs[bn], gsem.at[bn])
            return 0
        lax.fori_loop(0, CH // 4, gstep, 0)
        pltpu.make_async_copy(gbufs[(CH - 1) % 4], acc.at[dst_v.at[0, 0]],
                              ssem.at[(CH - 1) % 4]).wait()
        plsc.subcore_barrier()
        pltpu.sync_copy(acc.at[pl.ds(sid * RSUB, RSUB)],
                        out_hbm.at[cid, pl.ds(sid * RSUB, RSUB)])

        @pl.when(sid == 0)
        def _dtail():
            pltpu.sync_copy(acc.at[pl.ds(NS * RSUB, RTAIL)],
                            out_hbm.at[cid, pl.ds(NS * RSUB, RTAIL)])
    return k


@functools.cache
def _pool_kernel():
    rows_pt = NPAD // NW

    @functools.partial(
        pl.kernel,
        out_type=jax.ShapeDtypeStruct((NW, NG, D), jnp.float32),
        mesh=_sc_mesh(),
        compiler_params=pltpu.CompilerParams(needs_layout_passes=False),
        scratch_types=[
            pltpu.VMEM((rows_pt, D), jnp.float32),
            pltpu.VMEM((rows_pt,), jnp.int32),
            pltpu.VMEM((NG, D), jnp.float32),
        ],
    )
    def k(r_hbm, b_hbm, out_hbm, rows_v, b_v, acc_v):
        cid = lax.axis_index("c")
        sid = lax.axis_index("s")
        wid = sid * NC + cid
        neg1 = jnp.full((L,), -1.0, jnp.float32)

        def zrow(g, _):
            for v in range(D // L):
                acc_v[g, pl.ds(v * L, L)] = neg1
            return 0
        lax.fori_loop(0, NG, zrow, 0)
        pltpu.sync_copy(r_hbm.at[pl.ds(wid * rows_pt, rows_pt)], rows_v)
        pltpu.sync_copy(b_hbm.at[pl.ds(wid * rows_pt, rows_pt)], b_v)

        def row(i, _):
            gv = plsc.load_gather(b_v, [jnp.full((L,), i, jnp.int32)])
            g = gv[0]
            for v in range(D // L):
                sl = pl.ds(v * L, L)
                acc_v[g, sl] = jnp.maximum(acc_v[g, sl], rows_v[i, sl])
            return 0
        lax.fori_loop(0, rows_pt, row, 0)
        pltpu.sync_copy(acc_v, out_hbm.at[wid])
    return k


def kernel(x, edge_index, batch, dropout, edge_attr, device, glob_feat,
           params):
    p = params
    convs = [p['conv1']] + [p['hidden'][b][i] for b in range(6)
                            for i in range(2)]
    w1cat = jnp.concatenate([c['mw1'] for c in convs], axis=1)
    b1cat = jnp.concatenate([c['mb1'] for c in convs])[None, :]
    w2blk = jax.scipy.linalg.block_diag(*[c['mw2'] for c in convs])
    w2blk = jnp.pad(w2blk, ((0, 0), (0, 3)))
    b2cat = jnp.pad(jnp.concatenate([c['mb2'] for c in convs])[None, :],
                    ((0, 0), (0, 3)))

    w_all = _edge_mlp(edge_attr, w1cat, b1cat, w2blk, b2cat)

    src = edge_index[0].astype(jnp.int32)
    dst = edge_index[1].astype(jnp.int32)
    srcp = jnp.pad(src, (0, EP - EDG)).reshape(NW, CH, CK)
    dstp = jnp.pad(dst, (0, EP - EDG)).reshape(NW, CH, CK)
    dstp_d = jnp.pad(dst, (0, EP - EDG)).reshape(NW, DCH, DCK)
    w_pad = jnp.pad(w_all, ((0, EP - EDG), (0, 0)))

    degp = _deg_kernel()(w_pad.reshape(NW, DCH, DCK * 16), dstp_d)
    dinv, dinv2 = _dinv(degp)

    xl = _pre0(x, convs[0]['W'])
    s_acc = None
    out = None
    for c in range(NCONV):
        wcol = w_pad[:, c].reshape(NW, CH, CK)
        dcol = dinv[:, c]
        part = _msg_kernel()(xl, srcp, dstp, wcol, dcol)
        b_c = convs[c]['b'][None, :]
        if c == NCONV - 1:
            rfin = _final_asm(part, xl, dinv2, b_c, s_acc, c)
            bpad = jnp.pad(batch.astype(jnp.int32), (0, NPAD - N))
            poolp = _pool_kernel()(rfin, bpad)
            out = _logits(poolp, glob_feat, p['lin_W'],
                          p['lin_b'][None, :])
        elif c == 0:
            xl, s_acc = _pre_first(part, xl, dinv2, b_c,
                                   convs[c + 1]['W'], c)
        elif c % 2 == 0:
            xl, s_acc = _pre_bound(part, xl, dinv2, b_c,
                                   convs[c + 1]['W'], s_acc, c)
        else:
            xl = _pre_plain(part, xl, dinv2, b_c, convs[c + 1]['W'], c)
    return out


# speedup vs baseline: 1.0046x; 1.0046x over previous
import functools

import jax
import jax.numpy as jnp
from jax import lax
from jax.experimental import pallas as pl
from jax.experimental.pallas import tpu as pltpu
from jax.experimental.pallas import tpu_sc as plsc

N = 10000
EDG = 320000
D = 128
NCONV = 13
NG = 64
NC = 2
NS = 16
NW = NC * NS
L = 16
CK = 64
CH = 160
DCK = 128
DCH = 80
EP = NW * CH * CK
RSUB = 624
RTAIL = N - NS * RSUB
NPAD = NW * 320
HID = NCONV * 16


def _edge_mlp_body(ea_ref, w1_ref, b1_ref, w2_ref, b2_ref, o_ref):
    a = ea_ref[...]
    h = jnp.maximum(
        jnp.dot(a, w1_ref[...], preferred_element_type=jnp.float32) + b1_ref[...],
        0.0)
    z = jnp.dot(h, w2_ref[...], preferred_element_type=jnp.float32) + b2_ref[...]
    o_ref[...] = jax.nn.sigmoid(z)


def _edge_mlp(edge_attr, w1, b1, w2, b2):
    blk = 2000
    return pl.pallas_call(
        _edge_mlp_body,
        grid=(EDG // blk,),
        in_specs=[
            pl.BlockSpec((blk, 16), lambda i: (i, 0)),
            pl.BlockSpec((16, HID), lambda i: (0, 0)),
            pl.BlockSpec((1, HID), lambda i: (0, 0)),
            pl.BlockSpec((HID, 16), lambda i: (0, 0)),
            pl.BlockSpec((1, 16), lambda i: (0, 0)),
        ],
        out_specs=pl.BlockSpec((blk, 16), lambda i: (i, 0)),
        out_shape=jax.ShapeDtypeStruct((EDG, 16), jnp.float32),
    )(edge_attr, w1, b1, w2, b2)


def _dinv_body(degp_ref, dinv_ref, dinv2_ref):
    d = 1.0 + degp_ref[0][:, 0:16] + degp_ref[1][:, 0:16]
    dinv_ref[...] = lax.rsqrt(d)
    dinv2_ref[...] = 1.0 / d


def _dinv(degp):
    return pl.pallas_call(
        _dinv_body,
        out_shape=[
            jax.ShapeDtypeStruct((N, 16), jnp.float32),
            jax.ShapeDtypeStruct((N, 16), jnp.float32),
        ],
    )(degp)


def _pre0_body(x_ref, w_ref, o_ref):
    o_ref[...] = jnp.dot(x_ref[...], w_ref[...],
                         preferred_element_type=jnp.float32)


def _pre0(x, w):
    return pl.pallas_call(
        _pre0_body,
        out_shape=jax.ShapeDtypeStruct((N, D), jnp.float32),
    )(x, w)


def _asm(p_ref, xl_ref, d2_ref, b_ref, ci):
    d2c = d2_ref[...][:, ci:ci + 1]
    return p_ref[0] + p_ref[1] + d2c * xl_ref[...] + b_ref[...]


def _bn_relu_mm(xx, w_ref):
    m = jnp.mean(xx, axis=0, keepdims=True)
    xc = xx - m
    v = jnp.mean(xc * xc, axis=0, keepdims=True)
    h = jnp.maximum(xc * lax.rsqrt(v + 1e-5), 0.0)
    return jnp.dot(h, w_ref[...], preferred_element_type=jnp.float32)


def _pre_plain(part, xl, d2, b, wnext, ci):
    def body(p_ref, xl_ref, d2_ref, b_ref, w_ref, o_ref):
        o_ref[...] = _bn_relu_mm(_asm(p_ref, xl_ref, d2_ref, b_ref, ci), w_ref)
    return pl.pallas_call(
        body, out_shape=jax.ShapeDtypeStruct((N, D), jnp.float32),
    )(part, xl, d2, b, wnext)


def _pre_first(part, xl, d2, b, wnext, ci):
    def body(p_ref, xl_ref, d2_ref, b_ref, w_ref, o_ref, s_ref):
        x1 = _asm(p_ref, xl_ref, d2_ref, b_ref, ci)
        s_ref[...] = x1
        o_ref[...] = _bn_relu_mm(x1, w_ref)
    return pl.pallas_call(
        body,
        out_shape=[jax.ShapeDtypeStruct((N, D), jnp.float32),
                   jax.ShapeDtypeStruct((N, D), jnp.float32)],
    )(part, xl, d2, b, wnext)


def _pre_bound(part, xl, d2, b, wnext, s_in, ci):
    def body(p_ref, xl_ref, d2_ref, b_ref, w_ref, sin_ref, o_ref, sout_ref):
        xa = _asm(p_ref, xl_ref, d2_ref, b_ref, ci)
        s = xa + sin_ref[...]
        sout_ref[...] = sin_ref[...] + s
        o_ref[...] = _bn_relu_mm(s, w_ref)
    return pl.pallas_call(
        body,
        out_shape=[jax.ShapeDtypeStruct((N, D), jnp.float32),
                   jax.ShapeDtypeStruct((N, D), jnp.float32)],
    )(part, xl, d2, b, wnext, s_in)


def _final_asm(part, xl, d2, b, s_in, ci):
    def body(p_ref, xl_ref, d2_ref, b_ref, sin_ref, o_ref):
        xa = _asm(p_ref, xl_ref, d2_ref, b_ref, ci)
        s = xa + sin_ref[...]
        o_ref[pl.ds(0, N), :] = jnp.maximum(s, 0.0)
        o_ref[pl.ds(N, NPAD - N), :] = jnp.full((NPAD - N, D), -1.0,
                                                jnp.float32)
    return pl.pallas_call(
        body, out_shape=jax.ShapeDtypeStruct((NPAD, D), jnp.float32),
    )(part, xl, d2, b, s_in)


def _logits(poolp, glob, lw, lb):
    def body(pp_ref, g_ref, lw_ref, lb_ref, o_ref):
        pooled = jnp.max(pp_ref[...], axis=0)
        pooled = jnp.where(pooled < 0.0, 0.0, pooled)
        o_ref[...] = (
            jnp.dot(pooled, lw_ref[0:D, :], preferred_element_type=jnp.float32)
            + jnp.dot(g_ref[...], lw_ref[D:D + 16, :],
                      preferred_element_type=jnp.float32)
            + lb_ref[...])
    return pl.pallas_call(
        body, out_shape=jax.ShapeDtypeStruct((NG, 10), jnp.float32),
    )(poolp, glob, lw, lb)


@functools.cache
def _sc_mesh():
    return plsc.VectorSubcoreMesh(core_axis_name="c", subcore_axis_name="s")


@functools.cache
def _deg_kernel():
    @functools.partial(
        pl.kernel,
        out_type=jax.ShapeDtypeStruct((NC, N, D), jnp.float32),
        mesh=_sc_mesh(),
        compiler_params=pltpu.CompilerParams(needs_layout_passes=False),
        scratch_types=[
            pltpu.VMEM((DCH, DCK), jnp.int32),
            pltpu.VMEM((DCK * 16,), jnp.float32),
            pltpu.VMEM((DCK, D), jnp.float32),
            pltpu.VMEM((DCK, D), jnp.float32),
            pltpu.VMEM_SHARED((N, D), jnp.float32),
            pltpu.SemaphoreType.DMA((2,)),
        ],
    )
    def k(w16_hbm, dst_hbm, out_hbm, dst_v, wbuf, e0, e1, acc, ssem):
        ebufs = (e0, e1)
        cid = lax.axis_index("c")
        sid = lax.axis_index("s")
        wid = sid * NC + cid
        z16 = jnp.zeros((L,), jnp.float32)

        def zrow(i, _):
            for v in range(D // L):
                e0[i, pl.ds(v * L, L)] = z16
                e1[i, pl.ds(v * L, L)] = z16
            return 0
        lax.fori_loop(0, DCK, zrow, 0)
        for off, sz in ((0, 128), (128, 128), (256, 128), (384, 128),
                        (512, 112)):
            pltpu.sync_copy(e0.at[pl.ds(0, sz)],
                            acc.at[pl.ds(sid * RSUB + off, sz)])

        @pl.when(sid == 0)
        def _ztail():
            pltpu.sync_copy(e0.at[pl.ds(0, RTAIL)],
                            acc.at[pl.ds(NS * RSUB, RTAIL)])
        plsc.subcore_barrier()
        pltpu.sync_copy(dst_hbm.at[wid], dst_v)

        def gstep(g, _):
            for b in range(2):
                j = 2 * g + b
                pltpu.sync_copy(w16_hbm.at[wid, j], wbuf)

                @pl.when(j >= 2)
                def _ws():
                    pltpu.make_async_copy(ebufs[b], acc.at[dst_v.at[0]],
                                          ssem.at[b]).wait()

                def row(r, _):
                    ebufs[b][r, pl.ds(0, L)] = wbuf[pl.ds(r * 16, L)]
                    return 0
                lax.fori_loop(0, DCK, row, 0)
                pltpu.async_copy(ebufs[b], acc.at[dst_v.at[j]],
                                 ssem.at[b], add=True)
            return 0
        lax.fori_loop(0, DCH // 2, gstep, 0)
        for b in range(2):
            pltpu.make_async_copy(ebufs[b], acc.at[dst_v.at[0]],
                                  ssem.at[b]).wait()
        plsc.subcore_barrier()
        pltpu.sync_copy(acc.at[pl.ds(sid * RSUB, RSUB)],
                        out_hbm.at[cid, pl.ds(sid * RSUB, RSUB)])

        @pl.when(sid == 0)
        def _dtail():
            pltpu.sync_copy(acc.at[pl.ds(NS * RSUB, RTAIL)],
                            out_hbm.at[cid, pl.ds(NS * RSUB, RTAIL)])
    return k


@functools.cache
def _msg_kernel():
    @functools.partial(
        pl.kernel,
        out_type=jax.ShapeDtypeStruct((NC, N, D), jnp.float32),
        mesh=_sc_mesh(),
        compiler_params=pltpu.CompilerParams(needs_layout_passes=False),
        scratch_types=[
            pltpu.VMEM((2, 8, CK), jnp.int32),
            pltpu.VMEM((2, 8, CK), jnp.int32),
            pltpu.VMEM((2, 8, CK), jnp.float32),
            pltpu.VMEM((N,), jnp.float32),
            pltpu.VMEM((CK,), jnp.float32),
            pltpu.VMEM((CK, D), jnp.float32),
            pltpu.VMEM((CK, D), jnp.float32),
            pltpu.VMEM((CK, D), jnp.float32),
            pltpu.VMEM((CK, D), jnp.float32),
            pltpu.VMEM_SHARED((N, D), jnp.float32),
            pltpu.SemaphoreType.DMA((4,)),
            pltpu.SemaphoreType.DMA((4,)),
        ],
    )
    def k(xl_hbm, src_hbm, dst_hbm, w_hbm, dinv_hbm, out_hbm,
          src_v, dst_v, w_v, dinv_v, nrm_v, g0, g1, g2, g3, acc, gsem, ssem):
        gbufs = (g0, g1, g2, g3)
        cid = lax.axis_index("c")
        sid = lax.axis_index("s")
        wid = sid * NC + cid
        z16 = jnp.zeros((L,), jnp.float32)

        def zrow(i, _):
            for v in range(D // L):
                g0[i, pl.ds(v * L, L)] = z16
            return 0
        lax.fori_loop(0, CK, zrow, 0)
        for off in (0, 64, 128, 192, 256, 320, 384, 448, 512):
            pltpu.sync_copy(g0.at[pl.ds(0, 64)],
                            acc.at[pl.ds(sid * RSUB + off, 64)])
        pltpu.sync_copy(g0.at[pl.ds(0, 48)],
                        acc.at[pl.ds(sid * RSUB + 576, 48)])

        @pl.when(sid == 0)
        def _ztail():
            pltpu.sync_copy(g0.at[pl.ds(0, RTAIL)],
                            acc.at[pl.ds(NS * RSUB, RTAIL)])
        plsc.subcore_barrier()
        pltpu.sync_copy(dinv_hbm, dinv_v)

        def stage(grp_i, slot):
            base = pl.multiple_of(grp_i * 8, 8)
            pltpu.sync_copy(src_hbm.at[wid, pl.ds(base, 8)], src_v.at[slot])
            pltpu.sync_copy(dst_hbm.at[wid, pl.ds(base, 8)], dst_v.at[slot])
            pltpu.sync_copy(w_hbm.at[wid, pl.ds(base, 8)], w_v.at[slot])

        stage(0, 0)
        for b3 in range(2):
            pltpu.async_copy(xl_hbm.at[src_v.at[0, b3]], gbufs[b3],
                             gsem.at[b3])

        def gstep(g, _):
            for b in range(4):
                j = 4 * g + b
                pltpu.make_async_copy(xl_hbm.at[src_v.at[0, 0]], gbufs[b],
                                      gsem.at[b]).wait()
                slot = (j // 8) % 2
                r = j % 8
                for v in range(CK // L):
                    sl = pl.ds(v * L, L)
                    si = src_v[slot, r, sl]
                    di = dst_v[slot, r, sl]
                    nrm_v[sl] = (plsc.load_gather(dinv_v, [si])
                                 * w_v[slot, r, sl]
                                 * plsc.load_gather(dinv_v, [di]))

                def row(rr, _):
                    nb = plsc.load_gather(nrm_v,
                                          [jnp.full((L,), rr, jnp.int32)])
                    for v in range(D // L):
                        sl = pl.ds(v * L, L)
                        gbufs[b][rr, sl] = gbufs[b][rr, sl] * nb
                    return 0
                lax.fori_loop(0, CK, row, 0)
                pltpu.async_copy(gbufs[b], acc.at[dst_v.at[slot, r]],
                                 ssem.at[b], add=True)
                bn = (b + 2) % 4

                @pl.when(j >= 2)
                def _ws():
                    pltpu.make_async_copy(gbufs[bn], acc.at[dst_v.at[0, 0]],
                                          ssem.at[bn]).wait()
                j3 = j + 2
                if b == 2:
                    @pl.when((g % 2 == 1) & (j3 < CH))
                    def _st():
                        stage(j3 // 8, (j3 // 8) % 2)

                @pl.when(j3 < CH)
                def _ig():
                    pltpu.async_copy(
                        xl_hbm.at[src_v.at[(j3 // 8) % 2, j3 % 8]],
                        gbufs[bn], gsem.at[bn])
            return 0
        lax.fori_loop(0, CH // 4, gstep, 0)
        for jt in (CH - 2, CH - 1):
            pltpu.make_async_copy(gbufs[jt % 4], acc.at[dst_v.at[0, 0]],
                                  ssem.at[jt % 4]).wait()
        plsc.subcore_barrier()
        pltpu.sync_copy(acc.at[pl.ds(sid * RSUB, RSUB)],
                        out_hbm.at[cid, pl.ds(sid * RSUB, RSUB)])

        @pl.when(sid == 0)
        def _dtail():
            pltpu.sync_copy(acc.at[pl.ds(NS * RSUB, RTAIL)],
                            out_hbm.at[cid, pl.ds(NS * RSUB, RTAIL)])
    return k


@functools.cache
def _pool_kernel():
    rows_pt = NPAD // NW

    @functools.partial(
        pl.kernel,
        out_type=jax.ShapeDtypeStruct((NW, NG, D), jnp.float32),
        mesh=_sc_mesh(),
        compiler_params=pltpu.CompilerParams(needs_layout_passes=False),
        scratch_types=[
            pltpu.VMEM((rows_pt, D), jnp.float32),
            pltpu.VMEM((rows_pt,), jnp.int32),
            pltpu.VMEM((NG, D), jnp.float32),
        ],
    )
    def k(r_hbm, b_hbm, out_hbm, rows_v, b_v, acc_v):
        cid = lax.axis_index("c")
        sid = lax.axis_index("s")
        wid = sid * NC + cid
        neg1 = jnp.full((L,), -1.0, jnp.float32)

        def zrow(g, _):
            for v in range(D // L):
                acc_v[g, pl.ds(v * L, L)] = neg1
            return 0
        lax.fori_loop(0, NG, zrow, 0)
        pltpu.sync_copy(r_hbm.at[pl.ds(wid * rows_pt, rows_pt)], rows_v)
        pltpu.sync_copy(b_hbm.at[pl.ds(wid * rows_pt, rows_pt)], b_v)

        def row(i, _):
            gv = plsc.load_gather(b_v, [jnp.full((L,), i, jnp.int32)])
            g = gv[0]
            for v in range(D // L):
                sl = pl.ds(v * L, L)
                acc_v[g, sl] = jnp.maximum(acc_v[g, sl], rows_v[i, sl])
            return 0
        lax.fori_loop(0, rows_pt, row, 0)
        pltpu.sync_copy(acc_v, out_hbm.at[wid])
    return k


def kernel(x, edge_index, batch, dropout, edge_attr, device, glob_feat,
           params):
    p = params
    convs = [p['conv1']] + [p['hidden'][b][i] for b in range(6)
                            for i in range(2)]
    w1cat = jnp.concatenate([c['mw1'] for c in convs], axis=1)
    b1cat = jnp.concatenate([c['mb1'] for c in convs])[None, :]
    w2blk = jax.scipy.linalg.block_diag(*[c['mw2'] for c in convs])
    w2blk = jnp.pad(w2blk, ((0, 0), (0, 3)))
    b2cat = jnp.pad(jnp.concatenate([c['mb2'] for c in convs])[None, :],
                    ((0, 0), (0, 3)))

    w_all = _edge_mlp(edge_attr, w1cat, b1cat, w2blk, b2cat)

    src = edge_index[0].astype(jnp.int32)
    dst = edge_index[1].astype(jnp.int32)
    srcp = jnp.pad(src, (0, EP - EDG)).reshape(NW, CH, CK)
    dstp = jnp.pad(dst, (0, EP - EDG)).reshape(NW, CH, CK)
    dstp_d = jnp.pad(dst, (0, EP - EDG)).reshape(NW, DCH, DCK)
    w_pad = jnp.pad(w_all, ((0, EP - EDG), (0, 0)))

    degp = _deg_kernel()(w_pad.reshape(NW, DCH, DCK * 16), dstp_d)
    dinv, dinv2 = _dinv(degp)

    xl = _pre0(x, convs[0]['W'])
    s_acc = None
    out = None
    for c in range(NCONV):
        wcol = w_pad[:, c].reshape(NW, CH, CK)
        dcol = dinv[:, c]
        part = _msg_kernel()(xl, srcp, dstp, wcol, dcol)
        b_c = convs[c]['b'][None, :]
        if c == NCONV - 1:
            rfin = _final_asm(part, xl, dinv2, b_c, s_acc, c)
            bpad = jnp.pad(batch.astype(jnp.int32), (0, NPAD - N))
            poolp = _pool_kernel()(rfin, bpad)
            out = _logits(poolp, glob_feat, p['lin_W'],
                          p['lin_b'][None, :])
        elif c == 0:
            xl, s_acc = _pre_first(part, xl, dinv2, b_c,
                                   convs[c + 1]['W'], c)
        elif c % 2 == 0:
            xl, s_acc = _pre_bound(part, xl, dinv2, b_c,
                                   convs[c + 1]['W'], s_acc, c)
        else:
            xl = _pre_plain(part, xl, dinv2, b_c, convs[c + 1]['W'], c)
    return out
